# Initial kernel scaffold; baseline (speedup 1.0000x reference)
import jax
import jax.numpy as jnp
from jax.experimental import pallas as pl


def kernel(x, edge_index, W1, b1, W2, b2, W3, b3, W4, b4):
    raise NotImplementedError("write your pallas kernel here")


# speedup vs baseline: 10.2050x; 10.2050x over previous
import functools

import jax
import jax.numpy as jnp
from jax import lax
from jax.experimental import pallas as pl
from jax.experimental.pallas import tpu as pltpu
from jax.experimental.pallas import tpu_sc as plsc

LANES = 16
ROW_BLK = 1000
CHUNK = 80


def _sc_info():
    info = plsc.get_sparse_core_info()
    return info.num_cores, info.num_subcores


def _sc_degree(dst, zeros_n, n_nodes):
    nc, ns = _sc_info()
    e = dst.shape[0]
    epw = e // (nc * ns)
    nchunks = epw // CHUNK
    mesh = plsc.VectorSubcoreMesh(core_axis_name="c", subcore_axis_name="s")

    @functools.partial(
        pl.kernel,
        mesh=mesh,
        out_type=jax.ShapeDtypeStruct((nc, n_nodes), jnp.float32),
        scratch_types=[
            pltpu.VMEM((CHUNK,), jnp.int32),
            pltpu.VMEM((CHUNK,), jnp.float32),
            pltpu.VMEM_SHARED((n_nodes,), jnp.float32),
        ],
    )
    def k(dst_hbm, z_hbm, out_hbm, didx, ones_v, acc):
        c = lax.axis_index("c")
        s = lax.axis_index("s")
        one = jnp.ones((LANES,), jnp.float32)
        for j in range(CHUNK // LANES):
            ones_v[pl.ds(j * LANES, LANES)] = one

        @pl.when(s == 0)
        def _():
            pltpu.sync_copy(z_hbm, acc)

        plsc.subcore_barrier()
        base = (c * ns + s) * epw

        def body(i, carry):
            off = base + i * CHUNK
            pltpu.sync_copy(dst_hbm.at[pl.ds(off, CHUNK)], didx)
            pltpu.sync_copy(ones_v, acc.at[didx], add=True)
            return carry

        lax.fori_loop(0, nchunks, body, 0)
        plsc.subcore_barrier()

        @pl.when(s == 0)
        def _():
            pltpu.sync_copy(acc, out_hbm.at[c])

    return k(dst, zeros_n)


def _rows_per_tile(n, ns):
    return (-(-n // ns) + 7) // 8 * 8


def _sc_aggregate(g, src, dst, zeros):
    nc, ns = _sc_info()
    n, d = g.shape
    e = src.shape[0]
    epw = e // (nc * ns)
    nchunks = epw // CHUNK
    rpt = _rows_per_tile(n, ns)
    n_pad = rpt * ns
    mesh = plsc.VectorSubcoreMesh(core_axis_name="c", subcore_axis_name="s")

    @functools.partial(
        pl.kernel,
        mesh=mesh,
        out_type=jax.ShapeDtypeStruct((nc, n_pad, d), jnp.float32),
        scratch_types=[
            pltpu.VMEM((CHUNK,), jnp.int32),
            pltpu.VMEM((CHUNK,), jnp.int32),
            pltpu.VMEM((CHUNK, d), jnp.float32),
            pltpu.VMEM_SHARED((n_pad, d), jnp.float32),
            pltpu.SemaphoreType.DMA,
        ],
    )
    def k(g_hbm, src_hbm, dst_hbm, z_hbm, out_hbm, sidx, didx, rows, acc, sem):
        c = lax.axis_index("c")
        s = lax.axis_index("s")
        pltpu.sync_copy(z_hbm.at[pl.ds(s * rpt, rpt)], acc.at[pl.ds(s * rpt, rpt)])
        plsc.subcore_barrier()
        base = (c * ns + s) * epw

        def body(i, carry):
            off = base + i * CHUNK
            pltpu.sync_copy(src_hbm.at[pl.ds(off, CHUNK)], sidx)
            pltpu.sync_copy(dst_hbm.at[pl.ds(off, CHUNK)], didx)
            pltpu.async_copy(g_hbm.at[sidx], rows, sem).wait()
            pltpu.sync_copy(rows, acc.at[didx], add=True)
            return carry

        lax.fori_loop(0, nchunks, body, 0)
        plsc.subcore_barrier()
        pltpu.sync_copy(acc.at[pl.ds(s * rpt, rpt)], out_hbm.at[c, pl.ds(s * rpt, rpt)])

    return k(g, src, dst, zeros)


def _tc_dinv(deg_part, n):
    nw = deg_part.shape[0]

    def body(deg_ref, dinv_ref):
        ones = jnp.ones((nw, 1), jnp.float32)
        deg = lax.dot_general(
            deg_ref[...], ones, (((0,), (0,)), ((), ())),
            preferred_element_type=jnp.float32,
        )
        dinv_ref[...] = lax.rsqrt(deg + 1.0)

    return pl.pallas_call(
        body,
        out_shape=jax.ShapeDtypeStruct((n, 1), jnp.float32),
    )(deg_part)


def _tc_first(x, w1, dinv2):
    n, d = x.shape
    grid = n // ROW_BLK

    def body(x_ref, w_ref, dinv_ref, g_ref):
        g_ref[...] = (
            jnp.dot(x_ref[...], w_ref[...], preferred_element_type=jnp.float32)
            * dinv_ref[...]
        )

    return pl.pallas_call(
        body,
        grid=(grid,),
        in_specs=[
            pl.BlockSpec((ROW_BLK, d), lambda i: (i, 0)),
            pl.BlockSpec((d, d), lambda i: (0, 0)),
            pl.BlockSpec((ROW_BLK, 1), lambda i: (i, 0)),
        ],
        out_specs=pl.BlockSpec((ROW_BLK, d), lambda i: (i, 0)),
        out_shape=jax.ShapeDtypeStruct((n, d), jnp.float32),
    )(x, w1, dinv2)


def _tc_mid(p, g, dinv2, b_prev, w_next):
    n, d = g.shape
    nc, n_pad, _ = p.shape
    grid = n // ROW_BLK

    def body(p_ref, g_ref, dinv_ref, b_ref, w_ref, out_ref):
        dinv = dinv_ref[...]
        ps = jnp.sum(p_ref[...], axis=0)
        h = (ps + g_ref[...]) * dinv + b_ref[...]
        h = jnp.maximum(h, 0.0)
        out_ref[...] = (
            jnp.dot(h, w_ref[...], preferred_element_type=jnp.float32) * dinv
        )

    return pl.pallas_call(
        body,
        grid=(grid,),
        in_specs=[
            pl.BlockSpec((nc, ROW_BLK, d), lambda i: (0, i, 0)),
            pl.BlockSpec((ROW_BLK, d), lambda i: (i, 0)),
            pl.BlockSpec((ROW_BLK, 1), lambda i: (i, 0)),
            pl.BlockSpec((1, d), lambda i: (0, 0)),
            pl.BlockSpec((d, d), lambda i: (0, 0)),
        ],
        out_specs=pl.BlockSpec((ROW_BLK, d), lambda i: (i, 0)),
        out_shape=jax.ShapeDtypeStruct((n, d), jnp.float32),
    )(p, g, dinv2, b_prev, w_next)


def _tc_last(p, g, dinv2, b):
    n, d = g.shape
    nc, n_pad, _ = p.shape
    grid = n // ROW_BLK

    def body(p_ref, g_ref, dinv_ref, b_ref, out_ref):
        ps = jnp.sum(p_ref[...], axis=0)
        out_ref[...] = (ps + g_ref[...]) * dinv_ref[...] + b_ref[...]

    return pl.pallas_call(
        body,
        grid=(grid,),
        in_specs=[
            pl.BlockSpec((nc, ROW_BLK, d), lambda i: (0, i, 0)),
            pl.BlockSpec((ROW_BLK, d), lambda i: (i, 0)),
            pl.BlockSpec((ROW_BLK, 1), lambda i: (i, 0)),
            pl.BlockSpec((1, d), lambda i: (0, 0)),
        ],
        out_specs=pl.BlockSpec((ROW_BLK, d), lambda i: (i, 0)),
        out_shape=jax.ShapeDtypeStruct((n, d), jnp.float32),
    )(p, g, dinv2, b)


def kernel(x, edge_index, W1, b1, W2, b2, W3, b3, W4, b4):
    n, d = x.shape
    src = edge_index[0].astype(jnp.int32)
    dst = edge_index[1].astype(jnp.int32)
    _, ns = _sc_info()
    n_pad = _rows_per_tile(n, ns) * ns
    zeros = jnp.zeros((n_pad, d), jnp.float32)
    zeros_n = jnp.zeros((n,), jnp.float32)

    deg_part = _sc_degree(dst, zeros_n, n)
    dinv2 = _tc_dinv(deg_part, n)
    g = _tc_first(x, W1, dinv2)
    for b_prev, w_next in ((b1, W2), (b2, W3), (b3, W4)):
        p = _sc_aggregate(g, src, dst, zeros)
        g = _tc_mid(p, g, dinv2, b_prev.reshape(1, d), w_next)
    p = _sc_aggregate(g, src, dst, zeros)
    return _tc_last(p, g, dinv2, b4.reshape(1, d))


# speedup vs baseline: 19.0196x; 1.8638x over previous
import functools

import jax
import jax.numpy as jnp
from jax import lax
from jax.experimental import pallas as pl
from jax.experimental.pallas import tpu as pltpu
from jax.experimental.pallas import tpu_sc as plsc

LANES = 16
ROW_BLK = 1000
CHUNK = 80


def _sc_info():
    info = plsc.get_sparse_core_info()
    return info.num_cores, info.num_subcores


def _sc_degree(dst, zeros_n, n_nodes):
    nc, ns = _sc_info()
    e = dst.shape[0]
    epw = e // (nc * ns)
    nchunks = epw // CHUNK
    mesh = plsc.VectorSubcoreMesh(core_axis_name="c", subcore_axis_name="s")

    @functools.partial(
        pl.kernel,
        mesh=mesh,
        out_type=jax.ShapeDtypeStruct((nc, n_nodes), jnp.float32),
        scratch_types=[
            pltpu.VMEM((CHUNK,), jnp.int32),
            pltpu.VMEM((CHUNK,), jnp.float32),
            pltpu.VMEM_SHARED((n_nodes,), jnp.float32),
        ],
    )
    def k(dst_hbm, z_hbm, out_hbm, didx, ones_v, acc):
        c = lax.axis_index("c")
        s = lax.axis_index("s")
        one = jnp.ones((LANES,), jnp.float32)
        for j in range(CHUNK // LANES):
            ones_v[pl.ds(j * LANES, LANES)] = one

        @pl.when(s == 0)
        def _():
            pltpu.sync_copy(z_hbm, acc)

        plsc.subcore_barrier()
        base = (c * ns + s) * epw

        def body(i, carry):
            off = base + i * CHUNK
            pltpu.sync_copy(dst_hbm.at[pl.ds(off, CHUNK)], didx)
            pltpu.sync_copy(ones_v, acc.at[didx], add=True)
            return carry

        lax.fori_loop(0, nchunks, body, 0)
        plsc.subcore_barrier()

        @pl.when(s == 0)
        def _():
            pltpu.sync_copy(acc, out_hbm.at[c])

    return k(dst, zeros_n)


def _rows_per_tile(n, ns):
    return (-(-n // ns) + 7) // 8 * 8


def _sc_aggregate(g, edges4, zeros):
    nc, ns = _sc_info()
    n, d = g.shape
    nw, nchunks, _, chunk = edges4.shape
    rpt = _rows_per_tile(n, ns)
    n_pad = rpt * ns
    mesh = plsc.VectorSubcoreMesh(core_axis_name="c", subcore_axis_name="s")

    @functools.partial(
        pl.kernel,
        mesh=mesh,
        out_type=jax.ShapeDtypeStruct((nc, n_pad, d), jnp.float32),
        scratch_types=[
            pltpu.VMEM((2, 2, chunk), jnp.int32),
            pltpu.VMEM((2, chunk, d), jnp.float32),
            pltpu.VMEM_SHARED((n_pad, d), jnp.float32),
            pltpu.SemaphoreType.DMA((2,)),
            pltpu.SemaphoreType.DMA((2,)),
        ],
    )
    def k(g_hbm, e_hbm, z_hbm, out_hbm, idx, rows, acc, isem, gsem):
        c = lax.axis_index("c")
        s = lax.axis_index("s")
        w = c * ns + s
        pltpu.sync_copy(e_hbm.at[w, 0], idx.at[0])
        pltpu.async_copy(e_hbm.at[w, 1], idx.at[1], isem.at[1])
        pltpu.async_copy(g_hbm.at[idx.at[0, 0]], rows.at[0], gsem.at[0])
        pltpu.sync_copy(z_hbm.at[pl.ds(s * rpt, rpt)], acc.at[pl.ds(s * rpt, rpt)])
        plsc.subcore_barrier()

        def body(i, carry):
            b = lax.rem(i, 2)
            nb = lax.rem(i + 1, 2)

            @pl.when(i + 1 < nchunks)
            def _():
                pltpu.make_async_copy(e_hbm.at[w, 0], idx.at[nb], isem.at[nb]).wait()
                pltpu.async_copy(g_hbm.at[idx.at[nb, 0]], rows.at[nb], gsem.at[nb])

            pltpu.make_async_copy(g_hbm.at[idx.at[b, 0]], rows.at[b], gsem.at[b]).wait()
            pltpu.sync_copy(rows.at[b], acc.at[idx.at[b, 1]], add=True)

            @pl.when(i + 2 < nchunks)
            def _():
                pltpu.async_copy(e_hbm.at[w, i + 2], idx.at[b], isem.at[b])

            return carry

        lax.fori_loop(0, nchunks, body, 0)
        plsc.subcore_barrier()
        pltpu.sync_copy(acc.at[pl.ds(s * rpt, rpt)], out_hbm.at[c, pl.ds(s * rpt, rpt)])

    return k(g, edges4, zeros)


def _tc_dinv(deg_part, n):
    nw = deg_part.shape[0]

    def body(deg_ref, dinv_ref):
        ones = jnp.ones((nw, 1), jnp.float32)
        deg = lax.dot_general(
            deg_ref[...], ones, (((0,), (0,)), ((), ())),
            preferred_element_type=jnp.float32,
        )
        dinv_ref[...] = lax.rsqrt(deg + 1.0)

    return pl.pallas_call(
        body,
        out_shape=jax.ShapeDtypeStruct((n, 1), jnp.float32),
    )(deg_part)


def _tc_first(x, w1, dinv2):
    n, d = x.shape
    grid = n // ROW_BLK

    def body(x_ref, w_ref, dinv_ref, g_ref):
        g_ref[...] = (
            jnp.dot(x_ref[...], w_ref[...], preferred_element_type=jnp.float32)
            * dinv_ref[...]
        )

    return pl.pallas_call(
        body,
        grid=(grid,),
        in_specs=[
            pl.BlockSpec((ROW_BLK, d), lambda i: (i, 0)),
            pl.BlockSpec((d, d), lambda i: (0, 0)),
            pl.BlockSpec((ROW_BLK, 1), lambda i: (i, 0)),
        ],
        out_specs=pl.BlockSpec((ROW_BLK, d), lambda i: (i, 0)),
        out_shape=jax.ShapeDtypeStruct((n, d), jnp.float32),
    )(x, w1, dinv2)


def _tc_mid(p, g, dinv2, b_prev, w_next):
    n, d = g.shape
    nc, n_pad, _ = p.shape
    grid = n // ROW_BLK

    def body(p_ref, g_ref, dinv_ref, b_ref, w_ref, out_ref):
        dinv = dinv_ref[...]
        ps = jnp.sum(p_ref[...], axis=0)
        h = (ps + g_ref[...]) * dinv + b_ref[...]
        h = jnp.maximum(h, 0.0)
        out_ref[...] = (
            jnp.dot(h, w_ref[...], preferred_element_type=jnp.float32) * dinv
        )

    return pl.pallas_call(
        body,
        grid=(grid,),
        in_specs=[
            pl.BlockSpec((nc, ROW_BLK, d), lambda i: (0, i, 0)),
            pl.BlockSpec((ROW_BLK, d), lambda i: (i, 0)),
            pl.BlockSpec((ROW_BLK, 1), lambda i: (i, 0)),
            pl.BlockSpec((1, d), lambda i: (0, 0)),
            pl.BlockSpec((d, d), lambda i: (0, 0)),
        ],
        out_specs=pl.BlockSpec((ROW_BLK, d), lambda i: (i, 0)),
        out_shape=jax.ShapeDtypeStruct((n, d), jnp.float32),
    )(p, g, dinv2, b_prev, w_next)


def _tc_last(p, g, dinv2, b):
    n, d = g.shape
    nc, n_pad, _ = p.shape
    grid = n // ROW_BLK

    def body(p_ref, g_ref, dinv_ref, b_ref, out_ref):
        ps = jnp.sum(p_ref[...], axis=0)
        out_ref[...] = (ps + g_ref[...]) * dinv_ref[...] + b_ref[...]

    return pl.pallas_call(
        body,
        grid=(grid,),
        in_specs=[
            pl.BlockSpec((nc, ROW_BLK, d), lambda i: (0, i, 0)),
            pl.BlockSpec((ROW_BLK, d), lambda i: (i, 0)),
            pl.BlockSpec((ROW_BLK, 1), lambda i: (i, 0)),
            pl.BlockSpec((1, d), lambda i: (0, 0)),
        ],
        out_specs=pl.BlockSpec((ROW_BLK, d), lambda i: (i, 0)),
        out_shape=jax.ShapeDtypeStruct((n, d), jnp.float32),
    )(p, g, dinv2, b)


def kernel(x, edge_index, W1, b1, W2, b2, W3, b3, W4, b4):
    n, d = x.shape
    src = edge_index[0].astype(jnp.int32)
    dst = edge_index[1].astype(jnp.int32)
    nc, ns = _sc_info()
    nw = nc * ns
    e = src.shape[0]
    epw = e // nw
    edges4 = jnp.stack(
        [src.reshape(nw, epw // CHUNK, CHUNK), dst.reshape(nw, epw // CHUNK, CHUNK)],
        axis=2,
    )
    n_pad = _rows_per_tile(n, ns) * ns
    zeros = jnp.zeros((n_pad, d), jnp.float32)
    zeros_n = jnp.zeros((n,), jnp.float32)

    deg_part = _sc_degree(dst, zeros_n, n)
    dinv2 = _tc_dinv(deg_part, n)
    g = _tc_first(x, W1, dinv2)
    for b_prev, w_next in ((b1, W2), (b2, W3), (b3, W4)):
        p = _sc_aggregate(g, edges4, zeros)
        g = _tc_mid(p, g, dinv2, b_prev.reshape(1, d), w_next)
    p = _sc_aggregate(g, edges4, zeros)
    return _tc_last(p, g, dinv2, b4.reshape(1, d))
